# baseline (speedup 1.0000x reference)
import functools

import jax
import jax.numpy as jnp
from jax import lax
from jax.experimental import pallas as pl
from jax.experimental.pallas import tpu as pltpu
from jax.experimental.pallas import tpu_sc as plsc

N, D = 262144, 256
H = D // 2
NUM_CORES = 2
NUM_SUBCORES = 16
NW = NUM_CORES * NUM_SUBCORES
ROWS_PER_W = N // NW
RS = 64
RT = 64
NBUF = 3
A = 66
B = 62

_mesh = plsc.VectorSubcoreMesh(core_axis_name="c", subcore_axis_name="s")


@functools.partial(
    pl.kernel,
    mesh=_mesh,
    out_type=(
        jax.ShapeDtypeStruct((N, H), jnp.float32),
        jax.ShapeDtypeStruct((N, H), jnp.float32),
    ),
    scratch_types=[
        pltpu.MemorySpace.VMEM_SHARED((NUM_SUBCORES, NBUF, RS, D), jnp.float32),
        pltpu.VMEM((NBUF, RT, D), jnp.float32),
        pltpu.SemaphoreType.DMA,
        pltpu.SemaphoreType.DMA,
        pltpu.SemaphoreType.DMA,
        pltpu.SemaphoreType.DMA,
    ],
)
def _split_halves(inp_hbm, speed_hbm, dir_hbm, shared, tbuf, s_in, s_out, t_in, t_out):
    cid = lax.axis_index("c")
    sid = lax.axis_index("s")
    wid = sid * NUM_CORES + cid
    base = wid * ROWS_PER_W
    tbase = base + A * RS

    def make_path(buf_full, buf_left, buf_right, rows, in_sem, out_sem, chunks):
        def start_read(i, slot):
            pltpu.async_copy(inp_hbm.at[rows(i)], buf_full(slot), in_sem)

        def wait_read(i, slot):
            pltpu.make_async_copy(inp_hbm.at[rows(i)], buf_full(slot), in_sem).wait()

        def start_writes(i, slot):
            pltpu.async_copy(buf_left(slot), speed_hbm.at[rows(i)], out_sem)
            pltpu.async_copy(buf_right(slot), dir_hbm.at[rows(i)], out_sem)

        def wait_writes(i, slot):
            pltpu.make_async_copy(buf_left(slot), speed_hbm.at[rows(i)], out_sem).wait()
            pltpu.make_async_copy(buf_right(slot), dir_hbm.at[rows(i)], out_sem).wait()

        def prime():
            for j in range(min(NBUF, chunks)):
                start_read(j, j)

        def step(i):
            @pl.when(i < chunks)
            def _():
                slot = lax.rem(i, NBUF)

                @pl.when(i >= 1)
                def _():
                    prev_slot = lax.rem(i - 1, NBUF)
                    wait_writes(i - 1, prev_slot)

                    @pl.when(i - 1 + NBUF < chunks)
                    def _():
                        start_read(i - 1 + NBUF, prev_slot)

                wait_read(i, slot)
                start_writes(i, slot)

        def drain():
            wait_writes(chunks - 1, lax.rem(chunks - 1, NBUF))

        return prime, step, drain

    s_prime, s_step, s_drain = make_path(
        lambda slot: shared.at[sid, slot],
        lambda slot: shared.at[sid, slot, :, pl.ds(0, H)],
        lambda slot: shared.at[sid, slot, :, pl.ds(H, H)],
        lambda i: pl.ds(base + i * RS, RS),
        s_in, s_out, A,
    )
    t_prime, t_step, t_drain = make_path(
        lambda slot: tbuf.at[slot],
        lambda slot: tbuf.at[slot, :, pl.ds(0, H)],
        lambda slot: tbuf.at[slot, :, pl.ds(H, H)],
        lambda i: pl.ds(tbase + i * RT, RT),
        t_in, t_out, B,
    )

    s_prime()
    t_prime()

    def body(i, _):
        s_step(i)
        t_step(i)
        return 0

    lax.fori_loop(0, max(A, B), body, 0)
    s_drain()
    t_drain()


def kernel(inputs):
    return _split_halves(inputs)


# speedup vs baseline: 1.0374x; 1.0374x over previous
import functools

import jax
import jax.numpy as jnp
from jax import lax
from jax.experimental import pallas as pl
from jax.experimental.pallas import tpu as pltpu
from jax.experimental.pallas import tpu_sc as plsc

N, D = 262144, 256
H = D // 2
NUM_CORES = 2
NUM_SUBCORES = 16
NW = NUM_CORES * NUM_SUBCORES
ROWS_PER_W = N // NW
R = 128
CHUNKS = ROWS_PER_W // R
NBUF = 3

_mesh = plsc.VectorSubcoreMesh(core_axis_name="c", subcore_axis_name="s")


@functools.partial(
    pl.kernel,
    mesh=_mesh,
    out_type=(
        jax.ShapeDtypeStruct((N, H), jnp.float32),
        jax.ShapeDtypeStruct((N, H), jnp.float32),
    ),
    scratch_types=[
        pltpu.MemorySpace.VMEM_SHARED((NUM_SUBCORES, NBUF, R, D), jnp.float32),
        pltpu.SemaphoreType.DMA,
        pltpu.SemaphoreType.DMA,
    ],
)
def _split_halves(inp_hbm, speed_hbm, dir_hbm, shared, in_sem, out_sem):
    cid = lax.axis_index("c")
    sid = lax.axis_index("s")
    wid = sid * NUM_CORES + cid
    base = wid * ROWS_PER_W

    def rows(i):
        return pl.ds(base + i * R, R)

    def start_read(i, slot):
        pltpu.async_copy(inp_hbm.at[rows(i)], shared.at[sid, slot], in_sem)

    def wait_read(i, slot):
        pltpu.make_async_copy(inp_hbm.at[rows(i)], shared.at[sid, slot], in_sem).wait()

    def start_writes(i, slot):
        pltpu.async_copy(shared.at[sid, slot, :, pl.ds(0, H)], speed_hbm.at[rows(i)], out_sem)
        pltpu.async_copy(shared.at[sid, slot, :, pl.ds(H, H)], dir_hbm.at[rows(i)], out_sem)

    def wait_writes(i, slot):
        pltpu.make_async_copy(shared.at[sid, slot, :, pl.ds(0, H)], speed_hbm.at[rows(i)], out_sem).wait()
        pltpu.make_async_copy(shared.at[sid, slot, :, pl.ds(H, H)], dir_hbm.at[rows(i)], out_sem).wait()

    for j in range(NBUF):
        start_read(j, j)

    def body(i, _):
        slot = lax.rem(i, NBUF)

        @pl.when(i >= 1)
        def _():
            prev_slot = lax.rem(i - 1, NBUF)
            wait_writes(i - 1, prev_slot)

            @pl.when(i - 1 + NBUF < CHUNKS)
            def _():
                start_read(i - 1 + NBUF, prev_slot)

        wait_read(i, slot)
        start_writes(i, slot)
        return 0

    lax.fori_loop(0, CHUNKS, body, 0)
    wait_writes(CHUNKS - 1, lax.rem(CHUNKS - 1, NBUF))


def kernel(inputs):
    return _split_halves(inputs)
